# baseline (speedup 1.0000x reference)
import functools

import jax
import jax.numpy as jnp
from jax import lax
from jax.experimental import pallas as pl
from jax.experimental.pallas import tpu as pltpu
from jax.experimental.pallas import tpu_sc as plsc

_D = 128
_NC, _NS = 2, 16
_NW = _NC * _NS
_CK = 128
_CR = 136
_XW = 16


def _mm(a, b):
    return jnp.dot(a, b)


def _silu(v):
    return v * jax.nn.sigmoid(v)


def _iota16():
    return lax.broadcasted_iota(jnp.int32, (16,), 0)


def _sc_gather_body(atab, btab, xtab, idxr_hbm, idxc_hbm, outr, outc,
                    idxr_v, idxc_v, bufr, bufc, xr_v, xc_v,
                    bufr2, bufc2, xr_v2, xc_v2,
                    s1, s2, s3, s4, s5, s6, s7, s8, *, nchunks):
    cid = lax.axis_index("c")
    sid = lax.axis_index("s")
    wid = cid * _NS + sid
    pltpu.sync_copy(idxr_hbm.at[wid], idxr_v)
    pltpu.sync_copy(idxc_hbm.at[wid], idxc_v)

    def fire(j, bufr, bufc, xr_v, xc_v, s1, s2, s3, s4):
        ir = idxr_v.at[j]
        ic = idxc_v.at[j]
        pltpu.async_copy(atab.at[ir], bufr.at[pl.ds(0, _CK)], s1)
        pltpu.async_copy(btab.at[ic], bufc, s2)
        pltpu.async_copy(xtab.at[ir], xr_v, s3)
        pltpu.async_copy(xtab.at[ic], xc_v, s4)

    def finish(j, bufr, bufc, xr_v, xc_v, s1, s2, s3, s4):
        ir = idxr_v.at[j]
        ic = idxc_v.at[j]
        pltpu.make_async_copy(xtab.at[ir], xr_v, s3).wait()
        pltpu.make_async_copy(xtab.at[ic], xc_v, s4).wait()
        zero = jnp.zeros((16,), jnp.float32)
        for g in range(_CK // 16):
            rows = _iota16() + g * 16
            rel = []
            for v in range(3):
                cols = jnp.full((16,), v, jnp.int32)
                a = plsc.load_gather(xr_v, [rows, cols])
                b = plsc.load_gather(xc_v, [rows, cols])
                rel.append(a - b)
            d2 = rel[0] * rel[0] + rel[1] * rel[1] + rel[2] * rel[2]
            bufr[_CK + 0, pl.ds(g * 16, 16)] = rel[0]
            bufr[_CK + 1, pl.ds(g * 16, 16)] = rel[1]
            bufr[_CK + 2, pl.ds(g * 16, 16)] = rel[2]
            bufr[_CK + 3, pl.ds(g * 16, 16)] = d2
            for v in range(4):
                bufr[_CK + 4 + v, pl.ds(g * 16, 16)] = zero
        pltpu.make_async_copy(atab.at[ir], bufr.at[pl.ds(0, _CK)], s1).wait()
        pltpu.sync_copy(bufr, outr.at[pl.ds((wid * nchunks + j) * _CR, _CR)])
        pltpu.make_async_copy(btab.at[ic], bufc, s2).wait()
        pltpu.sync_copy(bufc, outc.at[pl.ds((wid * nchunks + j) * _CK, _CK)])

    seta = (bufr, bufc, xr_v, xc_v, s1, s2, s3, s4)
    setb = (bufr2, bufc2, xr_v2, xc_v2, s5, s6, s7, s8)
    fire(0, *seta)

    def step(i, carry):
        j0 = 2 * i
        fire(j0 + 1, *setb)
        finish(j0, *seta)

        @pl.when(j0 + 2 < nchunks)
        def _():
            fire(j0 + 2, *seta)

        finish(j0 + 1, *setb)
        return carry

    lax.fori_loop(0, nchunks // 2, step, 0)


def _sc_gather(atab, btab, xtab, idxr3, idxc3):
    nchunks = idxr3.shape[1]
    nck_tot = _NW * nchunks
    mesh = plsc.VectorSubcoreMesh(core_axis_name="c", subcore_axis_name="s")
    f = pl.kernel(
        functools.partial(_sc_gather_body, nchunks=nchunks),
        out_type=(jax.ShapeDtypeStruct((nck_tot * _CR, _D), jnp.float32),
                  jax.ShapeDtypeStruct((nck_tot * _CK, _D), jnp.float32)),
        mesh=mesh,
        scratch_types=[
            pltpu.VMEM((nchunks, _CK), jnp.int32),
            pltpu.VMEM((nchunks, _CK), jnp.int32),
            pltpu.VMEM((_CR, _D), jnp.float32),
            pltpu.VMEM((_CK, _D), jnp.float32),
            pltpu.VMEM((_CK, _XW), jnp.float32),
            pltpu.VMEM((_CK, _XW), jnp.float32),
            pltpu.VMEM((_CR, _D), jnp.float32),
            pltpu.VMEM((_CK, _D), jnp.float32),
            pltpu.VMEM((_CK, _XW), jnp.float32),
            pltpu.VMEM((_CK, _XW), jnp.float32),
        ] + [pltpu.SemaphoreType.DMA] * 8,
        compiler_params=pltpu.CompilerParams(use_tc_tiling_on_sc=False, needs_layout_passes=False),
    )
    return f(atab, btab, xtab, idxr3, idxc3)


def _edge_body(gr_ref, gc_ref, w1c_ref,
               w2_ref, b2_ref, c1_ref, cb1_ref, c2_ref, cb2_ref,
               p_ref, *, final: bool, nck: int):
    hr = jnp.concatenate(
        [gr_ref[pl.ds(k * _CR, _CK), :] for k in range(nck)], axis=0)
    hc = gc_ref[...]
    xt = jnp.concatenate(
        [gr_ref[pl.ds(k * _CR + _CK, 4), :] for k in range(nck)], axis=1)
    rel_t = xt[0:3]
    d2_t = xt[3:4]
    dist_t = jnp.sqrt(d2_t)
    sm_t = jnp.concatenate([rel_t, dist_t], axis=0)
    t = (hr + hc
         + lax.dot_general(sm_t, w1c_ref[...], (((0,), (0,)), ((), ())),
                           preferred_element_type=jnp.float32))
    t = _silu(t)
    m = _silu(jnp.dot(t, w2_ref[...], preferred_element_type=jnp.float32)
              + b2_ref[...])
    u = _silu(jnp.dot(m, c1_ref[...], preferred_element_type=jnp.float32)
              + cb1_ref[...])
    v_t = lax.dot_general(c2_ref[...], u, (((0,), (1,)), ((), ())),
                          preferred_element_type=jnp.float32) + cb2_ref[...]
    if final:
        s_t = v_t[0:3]
    else:
        cm_t = jnp.tanh(v_t[0:1])
        w_t = cm_t / (dist_t + 1e-8)
        s_t = w_t * rel_t
    b = hr.shape[0]
    s4_t = jnp.concatenate([s_t, jnp.ones((1, b), jnp.float32)], axis=0)
    for k in range(nck):
        if not final:
            p_ref[pl.ds(k * _CR, _CK), :] = m[k * _CK:(k + 1) * _CK, :]
        p_ref[pl.ds(k * _CR + _CK, 4), :] = s4_t[:, k * _CK:(k + 1) * _CK]
        p_ref[pl.ds(k * _CR + _CK + 4, 4), :] = jnp.zeros((4, _D), jnp.float32)


def _edge_pass(gr, gc, wts, final, nck=8):
    bc = gc.shape[0] // _CK
    assert bc % nck == 0
    grid = bc // nck
    full = lambda shape: pl.BlockSpec(shape, lambda i: (0, 0))
    (w1c, w2, b2, c1, cb1, c2, cb2) = wts
    return pl.pallas_call(
        functools.partial(_edge_body, final=final, nck=nck),
        grid=(grid,),
        in_specs=[
            pl.BlockSpec((nck * _CR, _D), lambda i: (i, 0)),
            pl.BlockSpec((nck * _CK, _D), lambda i: (i, 0)),
            full((4, _D)),
            full((_D, _D)), full((1, _D)),
            full((_D, _D)), full((1, _D)),
            full((_D, 8)), full((8, 1)),
        ],
        out_specs=pl.BlockSpec((nck * _CR, _D), lambda i: (i, 0)),
        out_shape=jax.ShapeDtypeStruct((bc * _CR, _D), jnp.float32),
    )(gr, gc, w1c, w2, b2, c1, cb1, c2, cb2)


def _sc_scatter_body(pay, idx_hbm, init1, init16, out1, out2,
                     idx_v, buf, stage, sem, acc, acc2, *, nchunks, nacc):
    cid = lax.axis_index("c")
    sid = lax.axis_index("s")
    wid = cid * _NS + sid
    zrows = nacc // _NS
    pltpu.sync_copy(init1.at[cid].at[pl.ds(sid * zrows, zrows)],
                    acc.at[pl.ds(sid * zrows, zrows)])
    pltpu.sync_copy(init16.at[cid].at[pl.ds(sid * zrows, zrows)],
                    acc2.at[pl.ds(sid * zrows, zrows)])
    zero = jnp.zeros((16,), jnp.float32)

    def zstep(i, carry):
        stage[i, :] = zero
        return carry

    lax.fori_loop(0, _CK, zstep, 0)
    plsc.subcore_barrier()
    pltpu.sync_copy(idx_hbm.at[wid], idx_v)
    iota = _iota16()

    def small_scatter(j, buf, off):
        for g in range(_CK // 16):
            rows = iota + g * 16
            for v in range(4):
                val = buf[off + v, pl.ds(g * 16, 16)]
                cols = jnp.full((16,), v, jnp.int32)
                plsc.store_scatter(stage, [rows, cols], val)
        pltpu.sync_copy(stage, acc2.at[idx_v.at[j]], add=True)

    def step(j, carry):
        pltpu.sync_copy(pay.at[pl.ds((wid * nchunks + j) * _CR, _CR)], buf)
        pltpu.sync_copy(buf.at[pl.ds(0, _CK)], acc.at[idx_v.at[j]], add=True)
        small_scatter(j, buf, _CK)
        return carry

    lax.fori_loop(0, nchunks, step, 0)
    plsc.subcore_barrier()
    pltpu.sync_copy(acc.at[pl.ds(sid * zrows, zrows)],
                    out1.at[cid].at[pl.ds(sid * zrows, zrows)])
    pltpu.sync_copy(acc2.at[pl.ds(sid * zrows, zrows)],
                    out2.at[cid].at[pl.ds(sid * zrows, zrows)])


def _sc_scatter_small_body(pay, idx_hbm, init16, out2,
                           idx_v, buf, stage, sem, acc2, *, nchunks, nacc):
    cid = lax.axis_index("c")
    sid = lax.axis_index("s")
    wid = cid * _NS + sid
    zrows = nacc // _NS
    pltpu.sync_copy(init16.at[cid].at[pl.ds(sid * zrows, zrows)],
                    acc2.at[pl.ds(sid * zrows, zrows)])
    zero = jnp.zeros((16,), jnp.float32)

    def zstep(i, carry):
        stage[i, :] = zero
        return carry

    lax.fori_loop(0, _CK, zstep, 0)
    plsc.subcore_barrier()
    pltpu.sync_copy(idx_hbm.at[wid], idx_v)
    iota = _iota16()

    def step(j, carry):
        pltpu.sync_copy(pay.at[pl.ds((wid * nchunks + j) * _CR + _CK, 8)], buf)
        for g in range(_CK // 16):
            rows = iota + g * 16
            for v in range(4):
                val = buf[v, pl.ds(g * 16, 16)]
                cols = jnp.full((16,), v, jnp.int32)
                plsc.store_scatter(stage, [rows, cols], val)
        pltpu.sync_copy(stage, acc2.at[idx_v.at[j]], add=True)
        return carry

    lax.fori_loop(0, nchunks, step, 0)
    plsc.subcore_barrier()
    pltpu.sync_copy(acc2.at[pl.ds(sid * zrows, zrows)],
                    out2.at[cid].at[pl.ds(sid * zrows, zrows)])


def _sc_scatter(payload, idx3, nacc, with_m=True, init=None):
    nchunks = idx3.shape[1]
    if init is None:
        init = (jnp.zeros((_NC, nacc, _D), jnp.float32),
                jnp.zeros((_NC, nacc, _XW), jnp.float32))
    init1, init16 = init
    if init16 is None:
        init16 = jnp.zeros((_NC, nacc, _XW), jnp.float32)
    mesh = plsc.VectorSubcoreMesh(core_axis_name="c", subcore_axis_name="s")
    if not with_m:
        f = pl.kernel(
            functools.partial(_sc_scatter_small_body, nchunks=nchunks,
                              nacc=nacc),
            out_type=jax.ShapeDtypeStruct((_NC, nacc, _XW), jnp.float32),
            mesh=mesh,
            scratch_types=[
                pltpu.VMEM((nchunks, _CK), jnp.int32),
                pltpu.VMEM((8, _D), jnp.float32),
                pltpu.VMEM((_CK, _XW), jnp.float32),
                pltpu.SemaphoreType.DMA,
                pltpu.VMEM_SHARED((nacc, _XW), jnp.float32),
            ],
            compiler_params=pltpu.CompilerParams(use_tc_tiling_on_sc=False, needs_layout_passes=False),
        )
        return None, f(payload, idx3, init16)
    f = pl.kernel(
        functools.partial(_sc_scatter_body, nchunks=nchunks, nacc=nacc),
        out_type=(jax.ShapeDtypeStruct((_NC, nacc, _D), jnp.float32),
                  jax.ShapeDtypeStruct((_NC, nacc, _XW), jnp.float32)),
        mesh=mesh,
        scratch_types=[
            pltpu.VMEM((nchunks, _CK), jnp.int32),
            pltpu.VMEM((_CR, _D), jnp.float32),
            pltpu.VMEM((_CK, _XW), jnp.float32),
            pltpu.SemaphoreType.DMA,
            pltpu.VMEM_SHARED((nacc, _D), jnp.float32),
            pltpu.VMEM_SHARED((nacc, _XW), jnp.float32),
        ],
        compiler_params=pltpu.CompilerParams(use_tc_tiling_on_sc=False, needs_layout_passes=False),
    )
    return f(payload, idx3, init1, init16)


def _layer_weights(lp):
    w1 = lp['msg1'][0]
    w1c = jnp.zeros((4, _D), jnp.float32).at[3].set(w1[2 * _D])
    c2 = jnp.zeros((_D, 8), jnp.float32).at[:, 0].set(lp['c2'][0][:, 0])
    proj = (w1[:_D], w1[_D:2 * _D], lp['msg1'][1])
    return proj, (w1c,
                  lp['msg2'][0], lp['msg2'][1][None],
                  lp['c1'][0], lp['c1'][1][None],
                  c2, jnp.zeros((8, 1), jnp.float32))


def _final_weights(p):
    w1 = p['em1'][0]
    w1c = jnp.concatenate([w1[2 * _D:2 * _D + 3], w1[2 * _D + 3:2 * _D + 4]],
                          axis=0)
    c2 = jnp.zeros((_D, 8), jnp.float32).at[:, :3].set(p['ec2'][0])
    cb2 = jnp.zeros((8, 1), jnp.float32).at[:3, 0].set(p['ec2'][1])
    proj = (w1[:_D], w1[_D:2 * _D], p['em1'][1])
    return proj, (w1c,
                  p['em2'][0], p['em2'][1][None],
                  p['ec1'][0], p['ec1'][1][None],
                  c2, cb2)


def _ln(v, g, b):
    mu = jnp.mean(v, axis=-1, keepdims=True)
    var = jnp.var(v, axis=-1, keepdims=True)
    return (v - mu) / jnp.sqrt(var + 1e-5) * g + b


def kernel(h, x, edge_index, params):
    p = params
    n, e = h.shape[0], edge_index.shape[1]
    nchunks = -(-e // (_NW * _CK))
    nchunks = -(-nchunks // 4) * 4
    ep = _NW * _CK * nchunks
    nacc = -(-(n + 1) // _D) * _D
    ntab = n + 16
    npadrows = nacc - n
    padi = jnp.arange(ep - e, dtype=jnp.int32)
    row = jnp.concatenate([edge_index[0], n + (padi % (ntab - n))])
    col = jnp.concatenate([edge_index[1], n + (padi % npadrows)])
    row3 = row.reshape(_NW, nchunks, _CK)
    col3 = col.reshape(_NW, nchunks, _CK)
    xtab_pad = jnp.zeros((ntab, _XW - 3), jnp.float32)
    tab_pad = jnp.zeros((ntab - n, _D), jnp.float32)

    wemb, bemb = p['emb']
    h = _mm(h, wemb) + bemb

    def tables(h, x, proj):
        w1a, w1b, b1 = proj
        atab = jnp.concatenate([_mm(h, w1a) + b1, tab_pad], axis=0)
        btab = jnp.concatenate([_mm(h, w1b), tab_pad], axis=0)
        xtab = jnp.concatenate(
            [jnp.concatenate([x, jnp.zeros((ntab - n, 3), jnp.float32)], axis=0),
             xtab_pad], axis=1)
        return atab, btab, xtab

    hc2 = nchunks // 2
    row3a, row3b = row3[:, :hc2], row3[:, hc2:]
    col3a, col3b = col3[:, :hc2], col3[:, hc2:]

    for lp in p['layers']:
        proj, wts = _layer_weights(lp)
        atab, btab, xtab = tables(h, x, proj)
        gra, gca = _sc_gather(atab, btab, xtab, row3a, col3a)
        paya = _edge_pass(gra, gca, wts, final=False)
        grb, gcb = _sc_gather(atab, btab, xtab, row3b, col3b)
        parta = _sc_scatter(paya, col3a, nacc)
        payb = _edge_pass(grb, gcb, wts, final=False)
        part1, part2 = _sc_scatter(payb, col3b, nacc, init=parta)
        agg = part1[0, :n] + part1[1, :n]
        s4 = part2[0, :n] + part2[1, :n]
        cu = s4[:, :3]
        deg = s4[:, 3:4]
        x = x + cu / (deg + 1.0)
        u1, ub1 = lp['u1']
        u2, ub2 = lp['u2']
        hu = _mm(_silu(_mm(h, u1[:_D]) + _mm(agg, u1[_D:]) + ub1), u2) + ub2
        g, bb = lp['ln']
        h = _ln(h + hu, g, bb)

    proj, wts = _final_weights(p)
    atab, btab, xtab = tables(h, x, proj)
    gra, gca = _sc_gather(atab, btab, xtab, row3a, col3a)
    paya = _edge_pass(gra, gca, wts, final=True)
    grb, gcb = _sc_gather(atab, btab, xtab, row3b, col3b)
    _, p2a = _sc_scatter(paya, col3a, nacc, with_m=False)
    payb = _edge_pass(grb, gcb, wts, final=True)
    _, part2 = _sc_scatter(payb, col3b, nacc, with_m=False,
                           init=(None, p2a))
    eps = part2[0, :n, :3] + part2[1, :n, :3]
    hw1, hb1 = p['eh1']
    hw2, hb2 = p['eh2']
    eps = eps + _mm(_silu(_mm(h, hw1[:_D]) + _mm(x, hw1[_D:]) + hb1), hw2) + hb2
    return (h, x, eps)


# speedup vs baseline: 1.0342x; 1.0342x over previous
import functools

import jax
import jax.numpy as jnp
from jax import lax
from jax.experimental import pallas as pl
from jax.experimental.pallas import tpu as pltpu
from jax.experimental.pallas import tpu_sc as plsc

_D = 128
_NC, _NS = 2, 16
_NW = _NC * _NS
_CK = 128
_CR = 136
_XW = 16


def _mm(a, b):
    return jnp.dot(a, b)


def _silu(v):
    return v * jax.nn.sigmoid(v)


def _iota16():
    return lax.broadcasted_iota(jnp.int32, (16,), 0)


def _sc_gather_body(atab, btab, xtab, idxr_hbm, idxc_hbm, outr, outc,
                    idxr_v, idxc_v, bufr, bufc, xr_v, xc_v,
                    bufr2, bufc2, xr_v2, xc_v2,
                    s1, s2, s3, s4, s5, s6, s7, s8, *, nchunks):
    cid = lax.axis_index("c")
    sid = lax.axis_index("s")
    wid = cid * _NS + sid
    pltpu.sync_copy(idxr_hbm.at[wid], idxr_v)
    pltpu.sync_copy(idxc_hbm.at[wid], idxc_v)

    def fire(j, bufr, bufc, xr_v, xc_v, s1, s2, s3, s4):
        ir = idxr_v.at[j]
        ic = idxc_v.at[j]
        pltpu.async_copy(atab.at[ir], bufr.at[pl.ds(0, _CK)], s1)
        pltpu.async_copy(btab.at[ic], bufc, s2)
        pltpu.async_copy(xtab.at[ir], xr_v, s3)
        pltpu.async_copy(xtab.at[ic], xc_v, s4)

    def finish(j, bufr, bufc, xr_v, xc_v, s1, s2, s3, s4):
        ir = idxr_v.at[j]
        ic = idxc_v.at[j]
        pltpu.make_async_copy(xtab.at[ir], xr_v, s3).wait()
        pltpu.make_async_copy(xtab.at[ic], xc_v, s4).wait()
        zero = jnp.zeros((16,), jnp.float32)
        for g in range(_CK // 16):
            rows = _iota16() + g * 16
            rel = []
            for v in range(3):
                cols = jnp.full((16,), v, jnp.int32)
                a = plsc.load_gather(xr_v, [rows, cols])
                b = plsc.load_gather(xc_v, [rows, cols])
                rel.append(a - b)
            d2 = rel[0] * rel[0] + rel[1] * rel[1] + rel[2] * rel[2]
            bufr[_CK + 0, pl.ds(g * 16, 16)] = rel[0]
            bufr[_CK + 1, pl.ds(g * 16, 16)] = rel[1]
            bufr[_CK + 2, pl.ds(g * 16, 16)] = rel[2]
            bufr[_CK + 3, pl.ds(g * 16, 16)] = d2
            for v in range(4):
                bufr[_CK + 4 + v, pl.ds(g * 16, 16)] = zero
        pltpu.make_async_copy(atab.at[ir], bufr.at[pl.ds(0, _CK)], s1).wait()
        pltpu.sync_copy(bufr, outr.at[pl.ds((wid * nchunks + j) * _CR, _CR)])
        pltpu.make_async_copy(btab.at[ic], bufc, s2).wait()
        pltpu.sync_copy(bufc, outc.at[pl.ds((wid * nchunks + j) * _CK, _CK)])

    seta = (bufr, bufc, xr_v, xc_v, s1, s2, s3, s4)
    setb = (bufr2, bufc2, xr_v2, xc_v2, s5, s6, s7, s8)
    fire(0, *seta)

    def step(i, carry):
        j0 = 2 * i
        fire(j0 + 1, *setb)
        finish(j0, *seta)

        @pl.when(j0 + 2 < nchunks)
        def _():
            fire(j0 + 2, *seta)

        finish(j0 + 1, *setb)
        return carry

    lax.fori_loop(0, nchunks // 2, step, 0)


def _sc_gather(atab, btab, xtab, idxr3, idxc3):
    nchunks = idxr3.shape[1]
    nck_tot = _NW * nchunks
    mesh = plsc.VectorSubcoreMesh(core_axis_name="c", subcore_axis_name="s")
    f = pl.kernel(
        functools.partial(_sc_gather_body, nchunks=nchunks),
        out_type=(jax.ShapeDtypeStruct((nck_tot * _CR, _D), jnp.float32),
                  jax.ShapeDtypeStruct((nck_tot * _CK, _D), jnp.float32)),
        mesh=mesh,
        scratch_types=[
            pltpu.VMEM((nchunks, _CK), jnp.int32),
            pltpu.VMEM((nchunks, _CK), jnp.int32),
            pltpu.VMEM((_CR, _D), jnp.float32),
            pltpu.VMEM((_CK, _D), jnp.float32),
            pltpu.VMEM((_CK, _XW), jnp.float32),
            pltpu.VMEM((_CK, _XW), jnp.float32),
            pltpu.VMEM((_CR, _D), jnp.float32),
            pltpu.VMEM((_CK, _D), jnp.float32),
            pltpu.VMEM((_CK, _XW), jnp.float32),
            pltpu.VMEM((_CK, _XW), jnp.float32),
        ] + [pltpu.SemaphoreType.DMA] * 8,
        compiler_params=pltpu.CompilerParams(use_tc_tiling_on_sc=False, needs_layout_passes=False),
    )
    return f(atab, btab, xtab, idxr3, idxc3)


def _edge_body(gr_ref, gc_ref, w1c_ref,
               w2_ref, b2_ref, c1_ref, cb1_ref, c2_ref, cb2_ref,
               p_ref, *, final: bool, nck: int):
    hr = jnp.concatenate(
        [gr_ref[pl.ds(k * _CR, _CK), :] for k in range(nck)], axis=0)
    hc = gc_ref[...]
    xt = jnp.concatenate(
        [gr_ref[pl.ds(k * _CR + _CK, 4), :] for k in range(nck)], axis=1)
    rel_t = xt[0:3]
    d2_t = xt[3:4]
    dist_t = jnp.sqrt(d2_t)
    sm_t = jnp.concatenate([rel_t, dist_t], axis=0)
    t = (hr + hc
         + lax.dot_general(sm_t, w1c_ref[...], (((0,), (0,)), ((), ())),
                           preferred_element_type=jnp.float32))
    t = _silu(t)
    m = _silu(jnp.dot(t, w2_ref[...], preferred_element_type=jnp.float32)
              + b2_ref[...])
    u = _silu(jnp.dot(m, c1_ref[...], preferred_element_type=jnp.float32)
              + cb1_ref[...])
    v_t = lax.dot_general(c2_ref[...], u, (((0,), (1,)), ((), ())),
                          preferred_element_type=jnp.float32) + cb2_ref[...]
    if final:
        s_t = v_t[0:3]
    else:
        cm_t = jnp.tanh(v_t[0:1])
        w_t = cm_t / (dist_t + 1e-8)
        s_t = w_t * rel_t
    b = hr.shape[0]
    s4_t = jnp.concatenate([s_t, jnp.ones((1, b), jnp.float32)], axis=0)
    for k in range(nck):
        if not final:
            p_ref[pl.ds(k * _CR, _CK), :] = m[k * _CK:(k + 1) * _CK, :]
        p_ref[pl.ds(k * _CR + _CK, 4), :] = s4_t[:, k * _CK:(k + 1) * _CK]
        p_ref[pl.ds(k * _CR + _CK + 4, 4), :] = jnp.zeros((4, _D), jnp.float32)


def _edge_pass(gr, gc, wts, final, nck=8):
    bc = gc.shape[0] // _CK
    assert bc % nck == 0
    grid = bc // nck
    full = lambda shape: pl.BlockSpec(shape, lambda i: (0, 0))
    (w1c, w2, b2, c1, cb1, c2, cb2) = wts
    return pl.pallas_call(
        functools.partial(_edge_body, final=final, nck=nck),
        grid=(grid,),
        in_specs=[
            pl.BlockSpec((nck * _CR, _D), lambda i: (i, 0)),
            pl.BlockSpec((nck * _CK, _D), lambda i: (i, 0)),
            full((4, _D)),
            full((_D, _D)), full((1, _D)),
            full((_D, _D)), full((1, _D)),
            full((_D, 8)), full((8, 1)),
        ],
        out_specs=pl.BlockSpec((nck * _CR, _D), lambda i: (i, 0)),
        out_shape=jax.ShapeDtypeStruct((bc * _CR, _D), jnp.float32),
    )(gr, gc, w1c, w2, b2, c1, cb1, c2, cb2)


def _sc_scatter_body(pay, idx_hbm, init1, init16, out1, out2,
                     idx_v, buf, stage, sem, acc, acc2, *, nchunks, nacc):
    cid = lax.axis_index("c")
    sid = lax.axis_index("s")
    wid = cid * _NS + sid
    zrows = nacc // _NS
    pltpu.sync_copy(init1.at[cid].at[pl.ds(sid * zrows, zrows)],
                    acc.at[pl.ds(sid * zrows, zrows)])
    pltpu.sync_copy(init16.at[cid].at[pl.ds(sid * zrows, zrows)],
                    acc2.at[pl.ds(sid * zrows, zrows)])
    zero = jnp.zeros((16,), jnp.float32)

    def zstep(i, carry):
        stage[i, :] = zero
        return carry

    lax.fori_loop(0, _CK, zstep, 0)
    plsc.subcore_barrier()
    pltpu.sync_copy(idx_hbm.at[wid], idx_v)
    iota = _iota16()

    def small_scatter(j, buf, off):
        for g in range(_CK // 16):
            rows = iota + g * 16
            for v in range(4):
                val = buf[off + v, pl.ds(g * 16, 16)]
                cols = jnp.full((16,), v, jnp.int32)
                plsc.store_scatter(stage, [rows, cols], val)
        pltpu.sync_copy(stage, acc2.at[idx_v.at[j]], add=True)

    def step(j, carry):
        pltpu.sync_copy(pay.at[pl.ds((wid * nchunks + j) * _CR, _CR)], buf)
        pltpu.sync_copy(buf.at[pl.ds(0, _CK)], acc.at[idx_v.at[j]], add=True)
        small_scatter(j, buf, _CK)
        return carry

    lax.fori_loop(0, nchunks, step, 0)
    plsc.subcore_barrier()
    pltpu.sync_copy(acc.at[pl.ds(sid * zrows, zrows)],
                    out1.at[cid].at[pl.ds(sid * zrows, zrows)])
    pltpu.sync_copy(acc2.at[pl.ds(sid * zrows, zrows)],
                    out2.at[cid].at[pl.ds(sid * zrows, zrows)])


def _sc_scatter_small_body(pay, idx_hbm, init16, out2,
                           idx_v, buf, stage, sem, acc2, *, nchunks, nacc):
    cid = lax.axis_index("c")
    sid = lax.axis_index("s")
    wid = cid * _NS + sid
    zrows = nacc // _NS
    pltpu.sync_copy(init16.at[cid].at[pl.ds(sid * zrows, zrows)],
                    acc2.at[pl.ds(sid * zrows, zrows)])
    zero = jnp.zeros((16,), jnp.float32)

    def zstep(i, carry):
        stage[i, :] = zero
        return carry

    lax.fori_loop(0, _CK, zstep, 0)
    plsc.subcore_barrier()
    pltpu.sync_copy(idx_hbm.at[wid], idx_v)
    iota = _iota16()

    def step(j, carry):
        pltpu.sync_copy(pay.at[pl.ds((wid * nchunks + j) * _CR + _CK, 8)], buf)
        for g in range(_CK // 16):
            rows = iota + g * 16
            for v in range(4):
                val = buf[v, pl.ds(g * 16, 16)]
                cols = jnp.full((16,), v, jnp.int32)
                plsc.store_scatter(stage, [rows, cols], val)
        pltpu.sync_copy(stage, acc2.at[idx_v.at[j]], add=True)
        return carry

    lax.fori_loop(0, nchunks, step, 0)
    plsc.subcore_barrier()
    pltpu.sync_copy(acc2.at[pl.ds(sid * zrows, zrows)],
                    out2.at[cid].at[pl.ds(sid * zrows, zrows)])


def _sc_scatter(payload, idx3, nacc, with_m=True, init=None):
    nchunks = idx3.shape[1]
    if init is None:
        init = (jnp.zeros((_NC, nacc, _D), jnp.float32),
                jnp.zeros((_NC, nacc, _XW), jnp.float32))
    init1, init16 = init
    if init16 is None:
        init16 = jnp.zeros((_NC, nacc, _XW), jnp.float32)
    if init1 is None:
        init1 = jnp.zeros((_NC, nacc, _D), jnp.float32)
    mesh = plsc.VectorSubcoreMesh(core_axis_name="c", subcore_axis_name="s")
    if not with_m:
        f = pl.kernel(
            functools.partial(_sc_scatter_small_body, nchunks=nchunks,
                              nacc=nacc),
            out_type=jax.ShapeDtypeStruct((_NC, nacc, _XW), jnp.float32),
            mesh=mesh,
            scratch_types=[
                pltpu.VMEM((nchunks, _CK), jnp.int32),
                pltpu.VMEM((8, _D), jnp.float32),
                pltpu.VMEM((_CK, _XW), jnp.float32),
                pltpu.SemaphoreType.DMA,
                pltpu.VMEM_SHARED((nacc, _XW), jnp.float32),
            ],
            compiler_params=pltpu.CompilerParams(use_tc_tiling_on_sc=False, needs_layout_passes=False),
        )
        return None, f(payload, idx3, init16)
    f = pl.kernel(
        functools.partial(_sc_scatter_body, nchunks=nchunks, nacc=nacc),
        out_type=(jax.ShapeDtypeStruct((_NC, nacc, _D), jnp.float32),
                  jax.ShapeDtypeStruct((_NC, nacc, _XW), jnp.float32)),
        mesh=mesh,
        scratch_types=[
            pltpu.VMEM((nchunks, _CK), jnp.int32),
            pltpu.VMEM((_CR, _D), jnp.float32),
            pltpu.VMEM((_CK, _XW), jnp.float32),
            pltpu.SemaphoreType.DMA,
            pltpu.VMEM_SHARED((nacc, _D), jnp.float32),
            pltpu.VMEM_SHARED((nacc, _XW), jnp.float32),
        ],
        compiler_params=pltpu.CompilerParams(use_tc_tiling_on_sc=False, needs_layout_passes=False),
    )
    return f(payload, idx3, init1, init16)


def _layer_weights(lp):
    w1 = lp['msg1'][0]
    w1c = jnp.zeros((4, _D), jnp.float32).at[3].set(w1[2 * _D])
    c2 = jnp.zeros((_D, 8), jnp.float32).at[:, 0].set(lp['c2'][0][:, 0])
    proj = (w1[:_D], w1[_D:2 * _D], lp['msg1'][1])
    return proj, (w1c,
                  lp['msg2'][0], lp['msg2'][1][None],
                  lp['c1'][0], lp['c1'][1][None],
                  c2, jnp.zeros((8, 1), jnp.float32))


def _final_weights(p):
    w1 = p['em1'][0]
    w1c = jnp.concatenate([w1[2 * _D:2 * _D + 3], w1[2 * _D + 3:2 * _D + 4]],
                          axis=0)
    c2 = jnp.zeros((_D, 8), jnp.float32).at[:, :3].set(p['ec2'][0])
    cb2 = jnp.zeros((8, 1), jnp.float32).at[:3, 0].set(p['ec2'][1])
    proj = (w1[:_D], w1[_D:2 * _D], p['em1'][1])
    return proj, (w1c,
                  p['em2'][0], p['em2'][1][None],
                  p['ec1'][0], p['ec1'][1][None],
                  c2, cb2)


def _ln(v, g, b):
    mu = jnp.mean(v, axis=-1, keepdims=True)
    var = jnp.var(v, axis=-1, keepdims=True)
    return (v - mu) / jnp.sqrt(var + 1e-5) * g + b


def kernel(h, x, edge_index, params):
    p = params
    n, e = h.shape[0], edge_index.shape[1]
    nchunks = -(-e // (_NW * _CK))
    nchunks = -(-nchunks // 8) * 8
    ep = _NW * _CK * nchunks
    nacc = -(-(n + 1) // _D) * _D
    ntab = n + 16
    npadrows = nacc - n
    padi = jnp.arange(ep - e, dtype=jnp.int32)
    row = jnp.concatenate([edge_index[0], n + (padi % (ntab - n))])
    col = jnp.concatenate([edge_index[1], n + (padi % npadrows)])
    row3 = row.reshape(_NW, nchunks, _CK)
    col3 = col.reshape(_NW, nchunks, _CK)
    xtab_pad = jnp.zeros((ntab, _XW - 3), jnp.float32)
    tab_pad = jnp.zeros((ntab - n, _D), jnp.float32)

    wemb, bemb = p['emb']
    h = _mm(h, wemb) + bemb

    def tables(h, x, proj):
        w1a, w1b, b1 = proj
        atab = jnp.concatenate([_mm(h, w1a) + b1, tab_pad], axis=0)
        btab = jnp.concatenate([_mm(h, w1b), tab_pad], axis=0)
        xtab = jnp.concatenate(
            [jnp.concatenate([x, jnp.zeros((ntab - n, 3), jnp.float32)], axis=0),
             xtab_pad], axis=1)
        return atab, btab, xtab

    nq = 4
    qc = nchunks // nq
    rowq = [row3[:, i * qc:(i + 1) * qc] for i in range(nq)]
    colq = [col3[:, i * qc:(i + 1) * qc] for i in range(nq)]

    for lp in p['layers']:
        proj, wts = _layer_weights(lp)
        atab, btab, xtab = tables(h, x, proj)
        pays = []
        for q in range(nq):
            grq, gcq = _sc_gather(atab, btab, xtab, rowq[q], colq[q])
            pays.append(_edge_pass(grq, gcq, wts, final=False))
        parts = None
        for q in range(nq):
            parts = _sc_scatter(pays[q], colq[q], nacc, init=parts)
        part1, part2 = parts
        agg = part1[0, :n] + part1[1, :n]
        s4 = part2[0, :n] + part2[1, :n]
        cu = s4[:, :3]
        deg = s4[:, 3:4]
        x = x + cu / (deg + 1.0)
        u1, ub1 = lp['u1']
        u2, ub2 = lp['u2']
        hu = _mm(_silu(_mm(h, u1[:_D]) + _mm(agg, u1[_D:]) + ub1), u2) + ub2
        g, bb = lp['ln']
        h = _ln(h + hu, g, bb)

    proj, wts = _final_weights(p)
    atab, btab, xtab = tables(h, x, proj)
    pays = []
    for q in range(nq):
        grq, gcq = _sc_gather(atab, btab, xtab, rowq[q], colq[q])
        pays.append(_edge_pass(grq, gcq, wts, final=True))
    p2 = None
    for q in range(nq):
        _, p2 = _sc_scatter(pays[q], colq[q], nacc, with_m=False,
                            init=(None, p2))
    eps = p2[0, :n, :3] + p2[1, :n, :3]
    hw1, hb1 = p['eh1']
    hw2, hb2 = p['eh2']
    eps = eps + _mm(_silu(_mm(h, hw1[:_D]) + _mm(x, hw1[_D:]) + hb1), hw2) + hb2
    return (h, x, eps)
